# baseline (speedup 1.0000x reference)
import functools

import jax
import jax.numpy as jnp
import numpy as np
from jax import lax
from jax.experimental import pallas as pl
from jax.experimental.pallas import tpu as pltpu
from jax.experimental.pallas import tpu_sc as plsc

N = 10000
E = 320000
DIN = 128
H = 128
OUT = 64

NC = 2
NS = 16
NW = NC * NS
CH = 64
CHN = 158
EPW = CH * CHN
EP = EPW * NW
TOTC = EP // CH
CA_MSG, CB_MSG = 106, 210
PADC = abs(CA_MSG - CB_MSG)
NP = 10112
RPT = NP // NS

K0, K1, K2, K3 = N, 8000, 6400, 5120

_INT_MIN = np.int32(-2147483648)
_MASK31 = np.int32(0x7FFFFFFF)
_TC_PARAMS = pltpu.CompilerParams(vmem_limit_bytes=120 * 1024 * 1024)


def _make_sc_scatter(d, ca, cb):
    mesh = plsc.VectorSubcoreMesh(
        core_axis_name="c", subcore_axis_name="s", num_cores=NC,
        num_subcores=NS)
    cmax = max(ca, cb)

    @functools.partial(
        pl.kernel,
        out_type=jax.ShapeDtypeStruct((NC, NP, d), jnp.float32),
        mesh=mesh,
        compiler_params=pltpu.CompilerParams(use_tc_tiling_on_sc=False),
        scratch_types=[
            pltpu.VMEM((cmax, CH), jnp.int32),
            pltpu.VMEM((cmax, CH), jnp.int32),
            pltpu.VMEM((CH, d), jnp.float32),
            pltpu.VMEM((CH, d), jnp.float32),
            pltpu.VMEM_SHARED((NP, d), jnp.float32),
            pltpu.SemaphoreType.DMA,
            pltpu.SemaphoreType.DMA,
        ],
    )
    def sc_scatter(values, src_r, dst_r, out, src_v, dst_v, rows0, rows1,
                   acc, sem0, sem1):
        cid = lax.axis_index("c")
        sid = lax.axis_index("s")
        if ca == cb:
            start = (cid * NS + sid) * ca
            cnt = ca
        else:
            start = jnp.where(cid == 0, sid * ca, NS * ca + sid * cb)
            cnt = jnp.where(cid == 0, ca, cb)
        pltpu.sync_copy(src_r.at[pl.ds(start, cmax)], src_v)
        pltpu.sync_copy(dst_r.at[pl.ds(start, cmax)], dst_v)

        def zb_body(i, carry):
            for j in range(d // 16):
                rows0[i, pl.ds(j * 16, 16)] = jnp.zeros((16,), jnp.float32)
            return carry

        lax.fori_loop(0, CH, zb_body, 0)
        row0 = sid * RPT
        nfull = RPT // CH
        rem = RPT - nfull * CH

        def zero_body(i, carry):
            pltpu.sync_copy(rows0, acc.at[pl.ds(row0 + i * CH, CH)])
            return carry

        lax.fori_loop(0, nfull, zero_body, 0)
        if rem:
            pltpu.sync_copy(rows0.at[pl.ds(0, rem)],
                            acc.at[pl.ds(row0 + nfull * CH, rem)])
        plsc.subcore_barrier()

        pltpu.async_copy(values.at[src_v.at[0]], rows0, sem0)

        def pair_body(i, carry):
            j = 2 * i
            pltpu.async_copy(values.at[src_v.at[j + 1]], rows1, sem1)
            pltpu.make_async_copy(values.at[src_v.at[j]], rows0, sem0).wait()
            pltpu.sync_copy(rows0, acc.at[dst_v.at[j]], add=True)
            pltpu.async_copy(values.at[src_v.at[j + 2]], rows0, sem0)
            pltpu.make_async_copy(
                values.at[src_v.at[j + 1]], rows1, sem1).wait()
            pltpu.sync_copy(rows1, acc.at[dst_v.at[j + 1]], add=True)
            return carry

        lax.fori_loop(0, (cnt - 2) // 2, pair_body, 0)
        pltpu.async_copy(values.at[src_v.at[cnt - 1]], rows1, sem1)
        pltpu.make_async_copy(
            values.at[src_v.at[cnt - 2]], rows0, sem0).wait()
        pltpu.sync_copy(rows0, acc.at[dst_v.at[cnt - 2]], add=True)
        pltpu.make_async_copy(
            values.at[src_v.at[cnt - 1]], rows1, sem1).wait()
        pltpu.sync_copy(rows1, acc.at[dst_v.at[cnt - 1]], add=True)

        plsc.subcore_barrier()
        pltpu.sync_copy(acc.at[pl.ds(row0, RPT)],
                        out.at[cid, pl.ds(row0, RPT)])

    return sc_scatter


_sc_deg = None
_sc_msg = None


def _sc_scatter_deg(values, src_r, dst_r):
    global _sc_deg
    if _sc_deg is None:
        _sc_deg = _make_sc_scatter(16, CHN, CHN)
    return _sc_deg(values, src_r, dst_r)


def _sc_scatter_msg(values, src_r, dst_r):
    global _sc_msg
    if _sc_msg is None:
        _sc_msg = _make_sc_scatter(H, CA_MSG, CB_MSG)
    return _sc_msg(values, src_r, dst_r)


def _tc_pre_body(x_ref, w1_ref, b1_ref, w2_ref, p_ref):
    h = jnp.dot(x_ref[...], w1_ref[...], preferred_element_type=jnp.float32)
    h = jnp.maximum(h + b1_ref[...], 0.0)
    p_ref[...] = jnp.dot(h, w2_ref[...], preferred_element_type=jnp.float32)


def _tc_pre(x, w1, b1, w2):
    return pl.pallas_call(
        _tc_pre_body,
        out_shape=jax.ShapeDtypeStruct((NP, H), jnp.float32),
    )(x, w1, b1, w2)


def _tc_dinv_body(da_ref, db_ref, alive_ref, p_ref, g_ref, dinv_ref):
    alive = alive_ref[...]
    deg = da_ref[:, 0:1] + db_ref[:, 0:1] + alive
    dinv = jnp.where(deg > 0.0, lax.rsqrt(deg), 0.0) * alive
    dinv_ref[...] = dinv
    g_ref[...] = dinv * p_ref[...]


def _tc_dinv(deg2, alive, p):
    return pl.pallas_call(
        _tc_dinv_body,
        out_shape=[
            jax.ShapeDtypeStruct((NP, H), jnp.float32),
            jax.ShapeDtypeStruct((NP, 1), jnp.float32),
        ],
    )(deg2[0], deg2[1], alive, p)


def _sortable_key(score, alive):
    kb = lax.bitcast_convert_type(score, jnp.int32)
    key = jnp.where(kb < 0, jnp.bitwise_xor(kb, _MASK31), kb)
    return jnp.where(alive > 0.0, key, _INT_MIN)


def _make_tc_search(k_new):
    def body(score2, alive2, t_ref, c_ref):
        key = _sortable_key(score2[...], alive2[...])

        def cnt_ge(t):
            return jnp.sum((key >= t).astype(jnp.int32))

        t0 = jnp.where(cnt_ge(jnp.int32(0)) >= k_new, jnp.int32(0),
                       _INT_MIN)

        def t_body(i, t):
            tp = t + lax.shift_left(jnp.int32(1), jnp.int32(30) - i)
            return jnp.where(cnt_ge(tp) >= k_new, tp, t)

        t = lax.fori_loop(0, 31, t_body, t0)
        eq = key == t
        need = jnp.int32(k_new) - jnp.sum((key > t).astype(jnp.int32))
        idx = (lax.broadcasted_iota(jnp.int32, (NP // 128, 128), 0) * 128
               + lax.broadcasted_iota(jnp.int32, (NP // 128, 128), 1))

        def c_body(i, c):
            cp = c + lax.shift_left(jnp.int32(1), jnp.int32(13) - i)
            n_below = jnp.sum((eq & (idx < cp)).astype(jnp.int32))
            return jnp.where(n_below <= need, cp, c)

        c = lax.fori_loop(0, 14, c_body, jnp.int32(0))
        t_ref[...] = jnp.full((1, 1), 0, jnp.int32) + t
        c_ref[...] = jnp.full((1, 1), 0, jnp.int32) + c

    return pl.pallas_call(
        body,
        compiler_params=_TC_PARAMS,
        out_shape=[
            jax.ShapeDtypeStruct((1, 1), jnp.int32),
            jax.ShapeDtypeStruct((1, 1), jnp.int32),
        ],
    )


def _make_tc_bn(k_prev):
    def body(msga, msgb, dinv, p, cb, bg, bb, pw, alive, h_ref, score_ref):
        dv = dinv[...]
        av = alive[...]
        out = dv * (msga[...] + msgb[...]) + dv * dv * p[...] + cb[...]
        m = jnp.sum(out * av, axis=0, keepdims=True) * (1.0 / k_prev)
        ctr = out - m
        v = jnp.sum(ctr * ctr * av, axis=0, keepdims=True) * (1.0 / k_prev)
        hbn = ctr / jnp.sqrt(v + 1e-5) * bg[...] + bb[...]
        h = jnp.maximum(hbn, 0.0) * av
        h_ref[...] = h
        pwv = pw[...]
        nrm = jnp.sqrt(jnp.sum(pwv * pwv))
        score_ref[...] = jnp.tanh(
            jnp.dot(h, pwv, preferred_element_type=jnp.float32) / nrm)

    return pl.pallas_call(
        body,
        compiler_params=_TC_PARAMS,
        out_shape=[
            jax.ShapeDtypeStruct((NP, H), jnp.float32),
            jax.ShapeDtypeStruct((NP, 1), jnp.float32),
        ],
    )


def _sel_readout(h, score, alive, t_ref, c_ref, k_new):
    key = _sortable_key(score[...], alive[...])
    t = t_ref[0, 0]
    c = c_ref[0, 0]
    idx = lax.broadcasted_iota(jnp.int32, (NP, 1), 0)
    keep = (key > t) | ((key == t) & (idx < c))
    anew = keep.astype(jnp.float32)
    hp = h[...] * score[...] * anew
    mx = jnp.max(jnp.where(keep, hp, -jnp.inf), axis=0, keepdims=True)
    mn = jnp.sum(hp, axis=0, keepdims=True) * (1.0 / k_new)
    return anew, hp, jnp.concatenate([mx, mn], axis=1)


def _make_tc_sel(k_new):
    def body(h, score, alive, t_ref, c_ref, wnext,
             anew_ref, mrows_ref, x_ref, pn_ref):
        anew, hp, x = _sel_readout(h, score, alive, t_ref, c_ref, k_new)
        anew_ref[...] = anew
        mrows_ref[...] = jnp.broadcast_to(anew, (NP, 16))
        x_ref[...] = x
        pn_ref[...] = jnp.dot(hp, wnext[...],
                              preferred_element_type=jnp.float32)

    return pl.pallas_call(
        body,
        compiler_params=_TC_PARAMS,
        out_shape=[
            jax.ShapeDtypeStruct((NP, 1), jnp.float32),
            jax.ShapeDtypeStruct((NP, 16), jnp.float32),
            jax.ShapeDtypeStruct((1, 2 * H), jnp.float32),
            jax.ShapeDtypeStruct((NP, H), jnp.float32),
        ],
    )


def _make_tc_sel_final(k_new):
    def body(h, score, alive, t_ref, c_ref, x1, x2, l1w, l1b, l2w, l2b,
             out_ref):
        _, _, x3 = _sel_readout(h, score, alive, t_ref, c_ref, k_new)
        z = x1[...] + x2[...] + x3
        z = jnp.maximum(
            jnp.dot(z, l1w[...], preferred_element_type=jnp.float32)
            + l1b[...], 0.0)
        out_ref[...] = (
            jnp.dot(z, l2w[...], preferred_element_type=jnp.float32)
            + l2b[...])

    return pl.pallas_call(
        body,
        compiler_params=_TC_PARAMS,
        out_shape=jax.ShapeDtypeStruct((1, OUT), jnp.float32),
    )


def kernel(x, edge_index, batch, W_in, b_in, conv1_W, conv1_b, bn1_g, bn1_b,
           pool1_w, conv2_W, conv2_b, bn2_g, bn2_b, pool2_w, conv3_W,
           conv3_b, bn3_g, bn3_b, pool3_w, lin1_W, lin1_b, lin2_W, lin2_b):
    f32 = jnp.float32
    pad_e = EP + PADC * CH - E
    src_r = jnp.concatenate(
        [edge_index[0], jnp.full((pad_e,), N, jnp.int32)]).reshape(
            TOTC + PADC, CH)
    dst_r = jnp.concatenate(
        [edge_index[1], jnp.full((pad_e,), N, jnp.int32)]).reshape(
            TOTC + PADC, CH)
    x_pad = jnp.zeros((NP, DIN), f32).at[:N].set(x)
    alive0 = (jnp.arange(NP, dtype=jnp.int32)[:, None] < N).astype(f32)
    m0rows = jnp.broadcast_to(alive0, (NP, 16))
    row = lambda a: a.reshape(1, -1)
    col = lambda a: a.reshape(-1, 1)

    two_d = lambda a: a.reshape(NP // 128, 128)

    p1 = _tc_pre(x_pad, W_in, row(b_in), conv1_W)
    deg1 = _sc_scatter_deg(m0rows, src_r, dst_r)
    g1, dinv1 = _tc_dinv(deg1, alive0, p1)
    msg1 = _sc_scatter_msg(g1, src_r, dst_r)
    h1, s1 = _make_tc_bn(K0)(
        msg1[0], msg1[1], dinv1, p1, row(conv1_b), row(bn1_g), row(bn1_b),
        col(pool1_w), alive0)
    t1, c1 = _make_tc_search(K1)(two_d(s1), two_d(alive0))
    alive1, m1rows, x1, p2 = _make_tc_sel(K1)(h1, s1, alive0, t1, c1,
                                              conv2_W)

    deg2 = _sc_scatter_deg(m1rows, src_r, dst_r)
    g2, dinv2 = _tc_dinv(deg2, alive1, p2)
    msg2 = _sc_scatter_msg(g2, src_r, dst_r)
    h2, s2 = _make_tc_bn(K1)(
        msg2[0], msg2[1], dinv2, p2, row(conv2_b), row(bn2_g), row(bn2_b),
        col(pool2_w), alive1)
    t2, c2 = _make_tc_search(K2)(two_d(s2), two_d(alive1))
    alive2, m2rows, x2, p3 = _make_tc_sel(K2)(h2, s2, alive1, t2, c2,
                                              conv3_W)

    deg3 = _sc_scatter_deg(m2rows, src_r, dst_r)
    g3, dinv3 = _tc_dinv(deg3, alive2, p3)
    msg3 = _sc_scatter_msg(g3, src_r, dst_r)
    h3, s3 = _make_tc_bn(K2)(
        msg3[0], msg3[1], dinv3, p3, row(conv3_b), row(bn3_g), row(bn3_b),
        col(pool3_w), alive2)
    t3, c3 = _make_tc_search(K3)(two_d(s3), two_d(alive2))
    return _make_tc_sel_final(K3)(
        h3, s3, alive2, t3, c3, x1, x2, lin1_W, row(lin1_b), lin2_W,
        row(lin2_b))


# speedup vs baseline: 1.1398x; 1.1398x over previous
import functools

import jax
import jax.numpy as jnp
import numpy as np
from jax import lax
from jax.experimental import pallas as pl
from jax.experimental.pallas import tpu as pltpu
from jax.experimental.pallas import tpu_sc as plsc

N = 10000
E = 320000
DIN = 128
H = 128
OUT = 64

NC = 2
NS = 16
NW = NC * NS
CH = 64
CHN = 158
EPW = CH * CHN
EP = EPW * NW
TOTC = EP // CH
CA_MSG, CB_MSG = 210, 106
PADC = abs(CA_MSG - CB_MSG)
NP = 10112
RPT = NP // NS

K0, K1, K2, K3 = N, 8000, 6400, 5120

_INT_MIN = np.int32(-2147483648)
_MASK31 = np.int32(0x7FFFFFFF)
_TC_PARAMS = pltpu.CompilerParams(vmem_limit_bytes=120 * 1024 * 1024)


def _make_sc_scatter(d, ca, cb):
    mesh = plsc.VectorSubcoreMesh(
        core_axis_name="c", subcore_axis_name="s", num_cores=NC,
        num_subcores=NS)
    cmax = max(ca, cb)

    @functools.partial(
        pl.kernel,
        out_type=jax.ShapeDtypeStruct((NC, NP, d), jnp.float32),
        mesh=mesh,
        compiler_params=pltpu.CompilerParams(use_tc_tiling_on_sc=False),
        scratch_types=[
            pltpu.VMEM((cmax, CH), jnp.int32),
            pltpu.VMEM((cmax, CH), jnp.int32),
            pltpu.VMEM((CH, d), jnp.float32),
            pltpu.VMEM((CH, d), jnp.float32),
            pltpu.VMEM_SHARED((NP, d), jnp.float32),
            pltpu.SemaphoreType.DMA,
            pltpu.SemaphoreType.DMA,
        ],
    )
    def sc_scatter(values, src_r, dst_r, out, src_v, dst_v, rows0, rows1,
                   acc, sem0, sem1):
        cid = lax.axis_index("c")
        sid = lax.axis_index("s")
        if ca == cb:
            start = (cid * NS + sid) * ca
            cnt = ca
        else:
            start = jnp.where(cid == 0, sid * ca, NS * ca + sid * cb)
            cnt = jnp.where(cid == 0, ca, cb)
        pltpu.sync_copy(src_r.at[pl.ds(start, cmax)], src_v)
        pltpu.sync_copy(dst_r.at[pl.ds(start, cmax)], dst_v)

        def zb_body(i, carry):
            for j in range(d // 16):
                rows0[i, pl.ds(j * 16, 16)] = jnp.zeros((16,), jnp.float32)
            return carry

        lax.fori_loop(0, CH, zb_body, 0)
        row0 = sid * RPT
        nfull = RPT // CH
        rem = RPT - nfull * CH

        def zero_body(i, carry):
            pltpu.sync_copy(rows0, acc.at[pl.ds(row0 + i * CH, CH)])
            return carry

        lax.fori_loop(0, nfull, zero_body, 0)
        if rem:
            pltpu.sync_copy(rows0.at[pl.ds(0, rem)],
                            acc.at[pl.ds(row0 + nfull * CH, rem)])
        plsc.subcore_barrier()

        pltpu.async_copy(values.at[src_v.at[0]], rows0, sem0)

        def pair_body(i, carry):
            j = 2 * i
            pltpu.async_copy(values.at[src_v.at[j + 1]], rows1, sem1)
            pltpu.make_async_copy(values.at[src_v.at[j]], rows0, sem0).wait()
            pltpu.sync_copy(rows0, acc.at[dst_v.at[j]], add=True)
            pltpu.async_copy(values.at[src_v.at[j + 2]], rows0, sem0)
            pltpu.make_async_copy(
                values.at[src_v.at[j + 1]], rows1, sem1).wait()
            pltpu.sync_copy(rows1, acc.at[dst_v.at[j + 1]], add=True)
            return carry

        lax.fori_loop(0, (cnt - 2) // 2, pair_body, 0)
        pltpu.async_copy(values.at[src_v.at[cnt - 1]], rows1, sem1)
        pltpu.make_async_copy(
            values.at[src_v.at[cnt - 2]], rows0, sem0).wait()
        pltpu.sync_copy(rows0, acc.at[dst_v.at[cnt - 2]], add=True)
        pltpu.make_async_copy(
            values.at[src_v.at[cnt - 1]], rows1, sem1).wait()
        pltpu.sync_copy(rows1, acc.at[dst_v.at[cnt - 1]], add=True)

        plsc.subcore_barrier()
        pltpu.sync_copy(acc.at[pl.ds(row0, RPT)],
                        out.at[cid, pl.ds(row0, RPT)])

    return sc_scatter


_sc_deg = None
_sc_msg = None


def _sc_scatter_deg(values, src_r, dst_r):
    global _sc_deg
    if _sc_deg is None:
        _sc_deg = _make_sc_scatter(16, CHN, CHN)
    return _sc_deg(values, src_r, dst_r)


def _sc_scatter_msg(values, src_r, dst_r):
    global _sc_msg
    if _sc_msg is None:
        _sc_msg = _make_sc_scatter(H, CA_MSG, CB_MSG)
    return _sc_msg(values, src_r, dst_r)


def _tc_pre_body(x_ref, w1_ref, b1_ref, w2_ref, p_ref):
    h = jnp.dot(x_ref[...], w1_ref[...], preferred_element_type=jnp.float32)
    h = jnp.maximum(h + b1_ref[...], 0.0)
    p_ref[...] = jnp.dot(h, w2_ref[...], preferred_element_type=jnp.float32)


def _tc_pre(x, w1, b1, w2):
    return pl.pallas_call(
        _tc_pre_body,
        out_shape=jax.ShapeDtypeStruct((NP, H), jnp.float32),
    )(x, w1, b1, w2)


def _tc_dinv_body(da_ref, db_ref, alive_ref, p_ref, g_ref, dinv_ref):
    alive = alive_ref[...]
    deg = da_ref[:, 0:1] + db_ref[:, 0:1] + alive
    dinv = jnp.where(deg > 0.0, lax.rsqrt(deg), 0.0) * alive
    dinv_ref[...] = dinv
    g_ref[...] = dinv * p_ref[...]


def _tc_dinv(deg2, alive, p):
    return pl.pallas_call(
        _tc_dinv_body,
        out_shape=[
            jax.ShapeDtypeStruct((NP, H), jnp.float32),
            jax.ShapeDtypeStruct((NP, 1), jnp.float32),
        ],
    )(deg2[0], deg2[1], alive, p)


def _sortable_key(score, alive):
    kb = lax.bitcast_convert_type(score, jnp.int32)
    key = jnp.where(kb < 0, jnp.bitwise_xor(kb, _MASK31), kb)
    return jnp.where(alive > 0.0, key, _INT_MIN)


def _make_tc_search(k_new):
    def body(score2, alive2, t_ref, c_ref):
        key = _sortable_key(score2[...], alive2[...])

        def cnt_ge(t):
            return jnp.sum((key >= t).astype(jnp.int32))

        t0 = jnp.where(cnt_ge(jnp.int32(0)) >= k_new, jnp.int32(0),
                       _INT_MIN)

        def t_body(i, t):
            tp = t + lax.shift_left(jnp.int32(1), jnp.int32(30) - i)
            return jnp.where(cnt_ge(tp) >= k_new, tp, t)

        t = lax.fori_loop(0, 31, t_body, t0)
        eq = key == t
        need = jnp.int32(k_new) - jnp.sum((key > t).astype(jnp.int32))
        idx = (lax.broadcasted_iota(jnp.int32, (NP // 128, 128), 0) * 128
               + lax.broadcasted_iota(jnp.int32, (NP // 128, 128), 1))

        def c_body(i, c):
            cp = c + lax.shift_left(jnp.int32(1), jnp.int32(13) - i)
            n_below = jnp.sum((eq & (idx < cp)).astype(jnp.int32))
            return jnp.where(n_below <= need, cp, c)

        c = lax.fori_loop(0, 14, c_body, jnp.int32(0))
        t_ref[...] = jnp.full((1, 1), 0, jnp.int32) + t
        c_ref[...] = jnp.full((1, 1), 0, jnp.int32) + c

    return pl.pallas_call(
        body,
        compiler_params=_TC_PARAMS,
        out_shape=[
            jax.ShapeDtypeStruct((1, 1), jnp.int32),
            jax.ShapeDtypeStruct((1, 1), jnp.int32),
        ],
    )


def _make_tc_bn(k_prev):
    def body(msga, msgb, dinv, p, cb, bg, bb, pw, alive, h_ref, score_ref):
        dv = dinv[...]
        av = alive[...]
        out = dv * (msga[...] + msgb[...]) + dv * dv * p[...] + cb[...]
        m = jnp.sum(out * av, axis=0, keepdims=True) * (1.0 / k_prev)
        ctr = out - m
        v = jnp.sum(ctr * ctr * av, axis=0, keepdims=True) * (1.0 / k_prev)
        hbn = ctr / jnp.sqrt(v + 1e-5) * bg[...] + bb[...]
        h = jnp.maximum(hbn, 0.0) * av
        h_ref[...] = h
        pwv = pw[...]
        nrm = jnp.sqrt(jnp.sum(pwv * pwv))
        score_ref[...] = jnp.tanh(
            jnp.dot(h, pwv, preferred_element_type=jnp.float32) / nrm)

    return pl.pallas_call(
        body,
        compiler_params=_TC_PARAMS,
        out_shape=[
            jax.ShapeDtypeStruct((NP, H), jnp.float32),
            jax.ShapeDtypeStruct((NP, 1), jnp.float32),
        ],
    )


def _sel_readout(h, score, alive, t_ref, c_ref, k_new):
    key = _sortable_key(score[...], alive[...])
    t = t_ref[0, 0]
    c = c_ref[0, 0]
    idx = lax.broadcasted_iota(jnp.int32, (NP, 1), 0)
    keep = (key > t) | ((key == t) & (idx < c))
    anew = keep.astype(jnp.float32)
    hp = h[...] * score[...] * anew
    mx = jnp.max(jnp.where(keep, hp, -jnp.inf), axis=0, keepdims=True)
    mn = jnp.sum(hp, axis=0, keepdims=True) * (1.0 / k_new)
    return anew, hp, jnp.concatenate([mx, mn], axis=1)


def _make_tc_sel(k_new):
    def body(h, score, alive, t_ref, c_ref, wnext,
             anew_ref, mrows_ref, x_ref, pn_ref):
        anew, hp, x = _sel_readout(h, score, alive, t_ref, c_ref, k_new)
        anew_ref[...] = anew
        mrows_ref[...] = jnp.broadcast_to(anew, (NP, 16))
        x_ref[...] = x
        pn_ref[...] = jnp.dot(hp, wnext[...],
                              preferred_element_type=jnp.float32)

    return pl.pallas_call(
        body,
        compiler_params=_TC_PARAMS,
        out_shape=[
            jax.ShapeDtypeStruct((NP, 1), jnp.float32),
            jax.ShapeDtypeStruct((NP, 16), jnp.float32),
            jax.ShapeDtypeStruct((1, 2 * H), jnp.float32),
            jax.ShapeDtypeStruct((NP, H), jnp.float32),
        ],
    )


def _make_tc_sel_final(k_new):
    def body(h, score, alive, t_ref, c_ref, x1, x2, l1w, l1b, l2w, l2b,
             out_ref):
        _, _, x3 = _sel_readout(h, score, alive, t_ref, c_ref, k_new)
        z = x1[...] + x2[...] + x3
        z = jnp.maximum(
            jnp.dot(z, l1w[...], preferred_element_type=jnp.float32)
            + l1b[...], 0.0)
        out_ref[...] = (
            jnp.dot(z, l2w[...], preferred_element_type=jnp.float32)
            + l2b[...])

    return pl.pallas_call(
        body,
        compiler_params=_TC_PARAMS,
        out_shape=jax.ShapeDtypeStruct((1, OUT), jnp.float32),
    )


def kernel(x, edge_index, batch, W_in, b_in, conv1_W, conv1_b, bn1_g, bn1_b,
           pool1_w, conv2_W, conv2_b, bn2_g, bn2_b, pool2_w, conv3_W,
           conv3_b, bn3_g, bn3_b, pool3_w, lin1_W, lin1_b, lin2_W, lin2_b):
    f32 = jnp.float32
    pad_e = EP + PADC * CH - E
    src_r = jnp.concatenate(
        [edge_index[0], jnp.full((pad_e,), N, jnp.int32)]).reshape(
            TOTC + PADC, CH)
    dst_r = jnp.concatenate(
        [edge_index[1], jnp.full((pad_e,), N, jnp.int32)]).reshape(
            TOTC + PADC, CH)
    x_pad = jnp.zeros((NP, DIN), f32).at[:N].set(x)
    alive0 = (jnp.arange(NP, dtype=jnp.int32)[:, None] < N).astype(f32)
    m0rows = jnp.broadcast_to(alive0, (NP, 16))
    row = lambda a: a.reshape(1, -1)
    col = lambda a: a.reshape(-1, 1)

    two_d = lambda a: a.reshape(NP // 128, 128)

    p1 = _tc_pre(x_pad, W_in, row(b_in), conv1_W)
    deg1 = _sc_scatter_deg(m0rows, src_r, dst_r)
    g1, dinv1 = _tc_dinv(deg1, alive0, p1)
    msg1 = _sc_scatter_msg(g1, src_r, dst_r)
    h1, s1 = _make_tc_bn(K0)(
        msg1[0], msg1[1], dinv1, p1, row(conv1_b), row(bn1_g), row(bn1_b),
        col(pool1_w), alive0)
    t1, c1 = _make_tc_search(K1)(two_d(s1), two_d(alive0))
    alive1, m1rows, x1, p2 = _make_tc_sel(K1)(h1, s1, alive0, t1, c1,
                                              conv2_W)

    deg2 = _sc_scatter_deg(m1rows, src_r, dst_r)
    g2, dinv2 = _tc_dinv(deg2, alive1, p2)
    msg2 = _sc_scatter_msg(g2, src_r, dst_r)
    h2, s2 = _make_tc_bn(K1)(
        msg2[0], msg2[1], dinv2, p2, row(conv2_b), row(bn2_g), row(bn2_b),
        col(pool2_w), alive1)
    t2, c2 = _make_tc_search(K2)(two_d(s2), two_d(alive1))
    alive2, m2rows, x2, p3 = _make_tc_sel(K2)(h2, s2, alive1, t2, c2,
                                              conv3_W)

    deg3 = _sc_scatter_deg(m2rows, src_r, dst_r)
    g3, dinv3 = _tc_dinv(deg3, alive2, p3)
    msg3 = _sc_scatter_msg(g3, src_r, dst_r)
    h3, s3 = _make_tc_bn(K2)(
        msg3[0], msg3[1], dinv3, p3, row(conv3_b), row(bn3_g), row(bn3_b),
        col(pool3_w), alive2)
    t3, c3 = _make_tc_search(K3)(two_d(s3), two_d(alive2))
    return _make_tc_sel_final(K3)(
        h3, s3, alive2, t3, c3, x1, x2, lin1_W, row(lin1_b), lin2_W,
        row(lin2_b))


# speedup vs baseline: 1.1856x; 1.0402x over previous
import functools

import jax
import jax.numpy as jnp
import numpy as np
from jax import lax
from jax.experimental import pallas as pl
from jax.experimental.pallas import tpu as pltpu
from jax.experimental.pallas import tpu_sc as plsc

N = 10000
E = 320000
DIN = 128
H = 128
OUT = 64

NC = 2
NS = 16
NW = NC * NS
CH = 64
CHN = 158
EPW = CH * CHN
EP = EPW * NW
TOTC = EP // CH
CA_MSG, CB_MSG = 232, 84
PADC = abs(CA_MSG - CB_MSG)
NP = 10112
RPT = NP // NS

K0, K1, K2, K3 = N, 8000, 6400, 5120

_INT_MIN = np.int32(-2147483648)
_MASK31 = np.int32(0x7FFFFFFF)
_TC_PARAMS = pltpu.CompilerParams(vmem_limit_bytes=120 * 1024 * 1024)


def _make_sc_scatter(d, ca, cb):
    mesh = plsc.VectorSubcoreMesh(
        core_axis_name="c", subcore_axis_name="s", num_cores=NC,
        num_subcores=NS)
    cmax = max(ca, cb)

    @functools.partial(
        pl.kernel,
        out_type=jax.ShapeDtypeStruct((NC, NP, d), jnp.float32),
        mesh=mesh,
        compiler_params=pltpu.CompilerParams(use_tc_tiling_on_sc=False),
        scratch_types=[
            pltpu.VMEM((cmax, CH), jnp.int32),
            pltpu.VMEM((cmax, CH), jnp.int32),
            pltpu.VMEM((CH, d), jnp.float32),
            pltpu.VMEM((CH, d), jnp.float32),
            pltpu.VMEM_SHARED((NP, d), jnp.float32),
            pltpu.SemaphoreType.DMA,
            pltpu.SemaphoreType.DMA,
        ],
    )
    def sc_scatter(values, src_r, dst_r, out, src_v, dst_v, rows0, rows1,
                   acc, sem0, sem1):
        cid = lax.axis_index("c")
        sid = lax.axis_index("s")
        if ca == cb:
            start = (cid * NS + sid) * ca
            cnt = ca
        else:
            start = jnp.where(cid == 0, sid * ca, NS * ca + sid * cb)
            cnt = jnp.where(cid == 0, ca, cb)
        pltpu.sync_copy(src_r.at[pl.ds(start, cmax)], src_v)
        pltpu.sync_copy(dst_r.at[pl.ds(start, cmax)], dst_v)

        def zb_body(i, carry):
            for j in range(d // 16):
                rows0[i, pl.ds(j * 16, 16)] = jnp.zeros((16,), jnp.float32)
            return carry

        lax.fori_loop(0, CH, zb_body, 0)
        row0 = sid * RPT
        nfull = RPT // CH
        rem = RPT - nfull * CH

        def zero_body(i, carry):
            pltpu.sync_copy(rows0, acc.at[pl.ds(row0 + i * CH, CH)])
            return carry

        lax.fori_loop(0, nfull, zero_body, 0)
        if rem:
            pltpu.sync_copy(rows0.at[pl.ds(0, rem)],
                            acc.at[pl.ds(row0 + nfull * CH, rem)])
        plsc.subcore_barrier()

        pltpu.async_copy(values.at[src_v.at[0]], rows0, sem0)

        def pair_body(i, carry):
            j = 2 * i
            pltpu.async_copy(values.at[src_v.at[j + 1]], rows1, sem1)
            pltpu.make_async_copy(values.at[src_v.at[j]], rows0, sem0).wait()
            pltpu.sync_copy(rows0, acc.at[dst_v.at[j]], add=True)
            pltpu.async_copy(values.at[src_v.at[j + 2]], rows0, sem0)
            pltpu.make_async_copy(
                values.at[src_v.at[j + 1]], rows1, sem1).wait()
            pltpu.sync_copy(rows1, acc.at[dst_v.at[j + 1]], add=True)
            return carry

        lax.fori_loop(0, (cnt - 2) // 2, pair_body, 0)
        pltpu.async_copy(values.at[src_v.at[cnt - 1]], rows1, sem1)
        pltpu.make_async_copy(
            values.at[src_v.at[cnt - 2]], rows0, sem0).wait()
        pltpu.sync_copy(rows0, acc.at[dst_v.at[cnt - 2]], add=True)
        pltpu.make_async_copy(
            values.at[src_v.at[cnt - 1]], rows1, sem1).wait()
        pltpu.sync_copy(rows1, acc.at[dst_v.at[cnt - 1]], add=True)

        plsc.subcore_barrier()
        pltpu.sync_copy(acc.at[pl.ds(row0, RPT)],
                        out.at[cid, pl.ds(row0, RPT)])

    return sc_scatter


_sc_deg = None
_sc_msg = None


def _sc_scatter_deg(values, src_r, dst_r):
    global _sc_deg
    if _sc_deg is None:
        _sc_deg = _make_sc_scatter(16, CHN, CHN)
    return _sc_deg(values, src_r, dst_r)


def _sc_scatter_msg(values, src_r, dst_r):
    global _sc_msg
    if _sc_msg is None:
        _sc_msg = _make_sc_scatter(H, CA_MSG, CB_MSG)
    return _sc_msg(values, src_r, dst_r)


def _tc_pre_body(x_ref, w1_ref, b1_ref, w2_ref, p_ref):
    h = jnp.dot(x_ref[...], w1_ref[...], preferred_element_type=jnp.float32)
    h = jnp.maximum(h + b1_ref[...], 0.0)
    p_ref[...] = jnp.dot(h, w2_ref[...], preferred_element_type=jnp.float32)


def _tc_pre(x, w1, b1, w2):
    return pl.pallas_call(
        _tc_pre_body,
        out_shape=jax.ShapeDtypeStruct((NP, H), jnp.float32),
    )(x, w1, b1, w2)


def _tc_dinv_body(da_ref, db_ref, alive_ref, p_ref, g_ref, dinv_ref):
    alive = alive_ref[...]
    deg = da_ref[:, 0:1] + db_ref[:, 0:1] + alive
    dinv = jnp.where(deg > 0.0, lax.rsqrt(deg), 0.0) * alive
    dinv_ref[...] = dinv
    g_ref[...] = dinv * p_ref[...]


def _tc_dinv(deg2, alive, p):
    return pl.pallas_call(
        _tc_dinv_body,
        out_shape=[
            jax.ShapeDtypeStruct((NP, H), jnp.float32),
            jax.ShapeDtypeStruct((NP, 1), jnp.float32),
        ],
    )(deg2[0], deg2[1], alive, p)


def _sortable_key(score, alive):
    kb = lax.bitcast_convert_type(score, jnp.int32)
    key = jnp.where(kb < 0, jnp.bitwise_xor(kb, _MASK31), kb)
    return jnp.where(alive > 0.0, key, _INT_MIN)


def _make_tc_search(k_new):
    def body(score2, alive2, t_ref, c_ref):
        key = _sortable_key(score2[...], alive2[...])

        def cnt_ge(t):
            return jnp.sum((key >= t).astype(jnp.int32))

        t0 = jnp.where(cnt_ge(jnp.int32(0)) >= k_new, jnp.int32(0),
                       _INT_MIN)

        def t_body(i, t):
            tp = t + lax.shift_left(jnp.int32(1), jnp.int32(30) - i)
            return jnp.where(cnt_ge(tp) >= k_new, tp, t)

        t = lax.fori_loop(0, 31, t_body, t0)
        eq = key == t
        need = jnp.int32(k_new) - jnp.sum((key > t).astype(jnp.int32))
        idx = (lax.broadcasted_iota(jnp.int32, (NP // 128, 128), 0) * 128
               + lax.broadcasted_iota(jnp.int32, (NP // 128, 128), 1))

        def c_body(i, c):
            cp = c + lax.shift_left(jnp.int32(1), jnp.int32(13) - i)
            n_below = jnp.sum((eq & (idx < cp)).astype(jnp.int32))
            return jnp.where(n_below <= need, cp, c)

        c = lax.fori_loop(0, 14, c_body, jnp.int32(0))
        t_ref[...] = jnp.full((1, 1), 0, jnp.int32) + t
        c_ref[...] = jnp.full((1, 1), 0, jnp.int32) + c

    return pl.pallas_call(
        body,
        compiler_params=_TC_PARAMS,
        out_shape=[
            jax.ShapeDtypeStruct((1, 1), jnp.int32),
            jax.ShapeDtypeStruct((1, 1), jnp.int32),
        ],
    )


def _make_tc_bn(k_prev):
    def body(msga, msgb, dinv, p, cb, bg, bb, pw, alive, h_ref, score_ref):
        dv = dinv[...]
        av = alive[...]
        out = dv * (msga[...] + msgb[...]) + dv * dv * p[...] + cb[...]
        m = jnp.sum(out * av, axis=0, keepdims=True) * (1.0 / k_prev)
        ctr = out - m
        v = jnp.sum(ctr * ctr * av, axis=0, keepdims=True) * (1.0 / k_prev)
        hbn = ctr / jnp.sqrt(v + 1e-5) * bg[...] + bb[...]
        h = jnp.maximum(hbn, 0.0) * av
        h_ref[...] = h
        pwv = pw[...]
        nrm = jnp.sqrt(jnp.sum(pwv * pwv))
        score_ref[...] = jnp.tanh(
            jnp.dot(h, pwv, preferred_element_type=jnp.float32) / nrm)

    return pl.pallas_call(
        body,
        compiler_params=_TC_PARAMS,
        out_shape=[
            jax.ShapeDtypeStruct((NP, H), jnp.float32),
            jax.ShapeDtypeStruct((NP, 1), jnp.float32),
        ],
    )


def _sel_readout(h, score, alive, t_ref, c_ref, k_new):
    key = _sortable_key(score[...], alive[...])
    t = t_ref[0, 0]
    c = c_ref[0, 0]
    idx = lax.broadcasted_iota(jnp.int32, (NP, 1), 0)
    keep = (key > t) | ((key == t) & (idx < c))
    anew = keep.astype(jnp.float32)
    hp = h[...] * score[...] * anew
    mx = jnp.max(jnp.where(keep, hp, -jnp.inf), axis=0, keepdims=True)
    mn = jnp.sum(hp, axis=0, keepdims=True) * (1.0 / k_new)
    return anew, hp, jnp.concatenate([mx, mn], axis=1)


def _make_tc_sel(k_new):
    def body(h, score, alive, t_ref, c_ref, wnext,
             anew_ref, mrows_ref, x_ref, pn_ref):
        anew, hp, x = _sel_readout(h, score, alive, t_ref, c_ref, k_new)
        anew_ref[...] = anew
        mrows_ref[...] = jnp.broadcast_to(anew, (NP, 16))
        x_ref[...] = x
        pn_ref[...] = jnp.dot(hp, wnext[...],
                              preferred_element_type=jnp.float32)

    return pl.pallas_call(
        body,
        compiler_params=_TC_PARAMS,
        out_shape=[
            jax.ShapeDtypeStruct((NP, 1), jnp.float32),
            jax.ShapeDtypeStruct((NP, 16), jnp.float32),
            jax.ShapeDtypeStruct((1, 2 * H), jnp.float32),
            jax.ShapeDtypeStruct((NP, H), jnp.float32),
        ],
    )


def _make_tc_sel_final(k_new):
    def body(h, score, alive, t_ref, c_ref, x1, x2, l1w, l1b, l2w, l2b,
             out_ref):
        _, _, x3 = _sel_readout(h, score, alive, t_ref, c_ref, k_new)
        z = x1[...] + x2[...] + x3
        z = jnp.maximum(
            jnp.dot(z, l1w[...], preferred_element_type=jnp.float32)
            + l1b[...], 0.0)
        out_ref[...] = (
            jnp.dot(z, l2w[...], preferred_element_type=jnp.float32)
            + l2b[...])

    return pl.pallas_call(
        body,
        compiler_params=_TC_PARAMS,
        out_shape=jax.ShapeDtypeStruct((1, OUT), jnp.float32),
    )


def kernel(x, edge_index, batch, W_in, b_in, conv1_W, conv1_b, bn1_g, bn1_b,
           pool1_w, conv2_W, conv2_b, bn2_g, bn2_b, pool2_w, conv3_W,
           conv3_b, bn3_g, bn3_b, pool3_w, lin1_W, lin1_b, lin2_W, lin2_b):
    f32 = jnp.float32
    pad_e = EP + PADC * CH - E
    src_r = jnp.concatenate(
        [edge_index[0], jnp.full((pad_e,), N, jnp.int32)]).reshape(
            TOTC + PADC, CH)
    dst_r = jnp.concatenate(
        [edge_index[1], jnp.full((pad_e,), N, jnp.int32)]).reshape(
            TOTC + PADC, CH)
    x_pad = jnp.zeros((NP, DIN), f32).at[:N].set(x)
    alive0 = (jnp.arange(NP, dtype=jnp.int32)[:, None] < N).astype(f32)
    m0rows = jnp.broadcast_to(alive0, (NP, 16))
    row = lambda a: a.reshape(1, -1)
    col = lambda a: a.reshape(-1, 1)

    two_d = lambda a: a.reshape(NP // 128, 128)

    p1 = _tc_pre(x_pad, W_in, row(b_in), conv1_W)
    deg1 = _sc_scatter_deg(m0rows, src_r, dst_r)
    g1, dinv1 = _tc_dinv(deg1, alive0, p1)
    msg1 = _sc_scatter_msg(g1, src_r, dst_r)
    h1, s1 = _make_tc_bn(K0)(
        msg1[0], msg1[1], dinv1, p1, row(conv1_b), row(bn1_g), row(bn1_b),
        col(pool1_w), alive0)
    t1, c1 = _make_tc_search(K1)(two_d(s1), two_d(alive0))
    alive1, m1rows, x1, p2 = _make_tc_sel(K1)(h1, s1, alive0, t1, c1,
                                              conv2_W)

    deg2 = _sc_scatter_deg(m1rows, src_r, dst_r)
    g2, dinv2 = _tc_dinv(deg2, alive1, p2)
    msg2 = _sc_scatter_msg(g2, src_r, dst_r)
    h2, s2 = _make_tc_bn(K1)(
        msg2[0], msg2[1], dinv2, p2, row(conv2_b), row(bn2_g), row(bn2_b),
        col(pool2_w), alive1)
    t2, c2 = _make_tc_search(K2)(two_d(s2), two_d(alive1))
    alive2, m2rows, x2, p3 = _make_tc_sel(K2)(h2, s2, alive1, t2, c2,
                                              conv3_W)

    deg3 = _sc_scatter_deg(m2rows, src_r, dst_r)
    g3, dinv3 = _tc_dinv(deg3, alive2, p3)
    msg3 = _sc_scatter_msg(g3, src_r, dst_r)
    h3, s3 = _make_tc_bn(K2)(
        msg3[0], msg3[1], dinv3, p3, row(conv3_b), row(bn3_g), row(bn3_b),
        col(pool3_w), alive2)
    t3, c3 = _make_tc_search(K3)(two_d(s3), two_d(alive2))
    return _make_tc_sel_final(K3)(
        h3, s3, alive2, t3, c3, x1, x2, lin1_W, row(lin1_b), lin2_W,
        row(lin2_b))
